# baseline (speedup 1.0000x reference)
import functools

import jax
import jax.numpy as jnp
from jax import lax
from jax.experimental import pallas as pl
from jax.experimental.pallas import tpu as pltpu


def _copy_logits_body(x_ref, wr_ref, out_ref, log_ref):
    x = x_ref[...]
    out_ref[...] = x
    log_ref[...] = jnp.dot(x, wr_ref[...], preferred_element_type=jnp.float32)


def _copy_logits(hs_flat, w_router, rows_per_blk=2048):
    n = hs_flat.shape[0]
    d = hs_flat.shape[1]
    grid = (n // rows_per_blk,)
    return pl.pallas_call(
        _copy_logits_body,
        grid=grid,
        in_specs=[
            pl.BlockSpec((rows_per_blk, d), lambda i: (i, 0)),
            pl.BlockSpec((d, 1), lambda i: (0, 0)),
        ],
        out_specs=[
            pl.BlockSpec((rows_per_blk, d), lambda i: (i, 0)),
            pl.BlockSpec((rows_per_blk, 1), lambda i: (i, 0)),
        ],
        out_shape=[
            jax.ShapeDtypeStruct((n, d), jnp.float32),
            jax.ShapeDtypeStruct((n, 1), jnp.float32),
        ],
    )(hs_flat, w_router)


def _cumsum_lanes(v, upper):
    n = v.shape[1]
    c = upper.shape[0]
    parts = []
    run = jnp.zeros((1, 1), jnp.float32)
    for i in range(n // c):
        blk = v[:, i * c:(i + 1) * c]
        cs = jnp.dot(blk, upper, preferred_element_type=jnp.float32) + run
        run = cs[:, c - 1:c]
        parts.append(cs)
    return jnp.concatenate(parts, axis=1)


def _topk_body(T, K, log_ref, idx_ref, gate_ref, bce_ref):
    b = pl.program_id(0)
    l = log_ref[0]
    bits = lax.bitcast_convert_type(l, jnp.int32)
    key = jnp.where(bits >= 0, bits, bits ^ jnp.int32(0x7FFFFFFF))

    def step(_, lohi):
        lo, hi = lohi
        x = lo ^ hi
        mid = (lo & hi) + (x >> 1) + (x & 1)
        cnt = jnp.sum((key >= mid).astype(jnp.int32))
        big = cnt >= K
        return jnp.where(big, mid, lo), jnp.where(big, hi, mid - 1)

    lo0 = jnp.int32(-2147483647 - 1)
    hi0 = jnp.int32(2147483647)
    t, _ = lax.fori_loop(0, 32, step, (lo0, hi0))

    cnt_gt = jnp.sum((key > t).astype(jnp.float32))
    m_extra = jnp.float32(K) - cnt_gt

    cdim = 1024
    io = lax.broadcasted_iota(jnp.int32, (cdim, cdim), 0)
    ic = lax.broadcasted_iota(jnp.int32, (cdim, cdim), 1)
    upper = (io <= ic).astype(jnp.float32)

    eq = (key == t).astype(jnp.float32)
    cum_eq = _cumsum_lanes(eq, upper)
    mask = jnp.where(key > t, 1.0, 0.0) + eq * (cum_eq <= m_extra)
    pos = _cumsum_lanes(mask, upper) * mask

    iota_f = lax.broadcasted_iota(jnp.int32, (1, T), 1).astype(jnp.float32)
    xt = jnp.concatenate([iota_f, l], axis=0)
    s_row = lax.broadcasted_iota(jnp.int32, (K, cdim), 0).astype(jnp.float32) + 1.0
    sel = jnp.zeros((2, K), jnp.float32)
    for i in range(T // cdim):
        pchunk = pos[:, i * cdim:(i + 1) * cdim]
        onehot = (jnp.broadcast_to(pchunk, (K, cdim)) == s_row).astype(jnp.float32)
        sel = sel + lax.dot_general(
            xt[:, i * cdim:(i + 1) * cdim], onehot,
            (((1,), (1,)), ((), ())), preferred_element_type=jnp.float32,
            precision=lax.Precision.HIGHEST)

    idx_f = sel[0:1, :]
    sel_l = sel[1:2, :]
    idx_ref[...] = (idx_f.astype(jnp.int32) + b * T).reshape(1, 1, K)
    gate_ref[...] = (1.0 / (1.0 + jnp.exp(-sel_l))).reshape(1, 1, K)

    softplus = jnp.maximum(l, 0.0) + jnp.log(1.0 + jnp.exp(-jnp.abs(l)))
    part = (jnp.sum(softplus, keepdims=True)
            - jnp.sum(sel_l, keepdims=True)).reshape(1, 1)

    @pl.when(b == 0)
    def _():
        bce_ref[...] = jnp.zeros((1, 1), jnp.float32)

    bce_ref[...] += part


def _topk(logits3, K):
    B, _, T = logits3.shape
    return pl.pallas_call(
        functools.partial(_topk_body, T, K),
        grid=(B,),
        in_specs=[pl.BlockSpec((1, 1, T), lambda b: (b, 0, 0))],
        out_specs=[
            pl.BlockSpec((1, 1, K), lambda b: (b, 0, 0)),
            pl.BlockSpec((1, 1, K), lambda b: (b, 0, 0)),
            pl.BlockSpec((1, 1), lambda b: (0, 0)),
        ],
        out_shape=[
            jax.ShapeDtypeStruct((B, 1, K), jnp.int32),
            jax.ShapeDtypeStruct((B, 1, K), jnp.float32),
            jax.ShapeDtypeStruct((1, 1), jnp.float32),
        ],
    )(logits3)


def _gather(table, idx, G=8):
    n, d = table.shape
    m = idx.shape[0]
    table3 = table.reshape(n, 1, d)

    def body(idx_sref, t_ref, out_ref):
        del idx_sref
        out_ref[...] = t_ref[...]

    gspec = pltpu.PrefetchScalarGridSpec(
        num_scalar_prefetch=1,
        grid=(m,),
        in_specs=[pl.BlockSpec((1, 1, d), lambda i, idx_s: (idx_s[i], 0, 0))],
        out_specs=pl.BlockSpec((1, 1, d), lambda i, idx_s: (i, 0, 0)),
    )
    return pl.pallas_call(
        body, grid_spec=gspec,
        out_shape=jax.ShapeDtypeStruct((m, 1, d), jnp.float32),
    )(idx, table3).reshape(m, d)


def _scatter_body(idx_sref, base_ref, rows_ref, out_ref):
    del idx_sref, base_ref
    out_ref[...] = rows_ref[...]


def _tc_scatter(out_copy, idx, rows):
    n, d = out_copy.shape
    m = rows.shape[0]
    gspec = pltpu.PrefetchScalarGridSpec(
        num_scalar_prefetch=1,
        grid=(m,),
        in_specs=[
            pl.BlockSpec(memory_space=pltpu.HBM),
            pl.BlockSpec((1, 1, d), lambda i, idx_s: (i, 0, 0)),
        ],
        out_specs=pl.BlockSpec((1, 1, d), lambda i, idx_s: (idx_s[i], 0, 0)),
    )
    return pl.pallas_call(
        _scatter_body, grid_spec=gspec,
        out_shape=jax.ShapeDtypeStruct((n, 1, d), jnp.float32),
        input_output_aliases={1: 0},
    )(idx, out_copy.reshape(n, 1, d), rows.reshape(m, 1, d))


def _attn_body(H, Dh, x_ref, ln1_ref, wq_ref, wk_ref, wv_ref, wo_ref, h1_ref):
    x = x_ref[0]
    h = x * lax.rsqrt(jnp.mean(x * x, axis=-1, keepdims=True) + 1e-6) * ln1_ref[...]
    K = x.shape[0]
    r_io = lax.broadcasted_iota(jnp.int32, (K, K), 0)
    c_io = lax.broadcasted_iota(jnp.int32, (K, K), 1)
    causal = c_io <= r_io
    scale = 1.0 / (Dh ** 0.5)
    outs = []
    for hd in range(H):
        sl = slice(hd * Dh, (hd + 1) * Dh)
        q = jnp.dot(h, wq_ref[:, sl], preferred_element_type=jnp.float32)
        kk = jnp.dot(h, wk_ref[:, sl], preferred_element_type=jnp.float32)
        v = jnp.dot(h, wv_ref[:, sl], preferred_element_type=jnp.float32)
        s = lax.dot_general(q, kk, (((1,), (1,)), ((), ())),
                            preferred_element_type=jnp.float32) * scale
        s = jnp.where(causal, s, jnp.float32(-1e9))
        m = jnp.max(s, axis=-1, keepdims=True)
        e = jnp.exp(s - m)
        p = e / jnp.sum(e, axis=-1, keepdims=True)
        outs.append(jnp.dot(p, v, preferred_element_type=jnp.float32))
    attn = jnp.concatenate(outs, axis=1)
    h1_ref[0] = x + jnp.dot(attn, wo_ref[...], preferred_element_type=jnp.float32)


def _attention(x_sel, ln1, wq, wk, wv, wo, H, Dh):
    B, K, D = x_sel.shape
    wspec = pl.BlockSpec((D, D), lambda b: (0, 0))
    return pl.pallas_call(
        functools.partial(_attn_body, H, Dh),
        grid=(B,),
        in_specs=[
            pl.BlockSpec((1, K, D), lambda b: (b, 0, 0)),
            pl.BlockSpec((1, D), lambda b: (0, 0)),
            wspec, wspec, wspec, wspec,
        ],
        out_specs=pl.BlockSpec((1, K, D), lambda b: (b, 0, 0)),
        out_shape=jax.ShapeDtypeStruct((B, K, D), jnp.float32),
    )(x_sel, ln1.reshape(1, D), wq, wk, wv, wo)


def _mlp_body(NF, h1_ref, x_ref, g_ref, ln2_ref, wg_ref, wu_ref, wd_ref,
              new_ref, m_s, acc_s):
    f = pl.program_id(1)

    @pl.when(f == 0)
    def _():
        h1 = h1_ref[0]
        m_s[...] = h1 * lax.rsqrt(jnp.mean(h1 * h1, axis=-1, keepdims=True)
                                  + 1e-6) * ln2_ref[...]
        acc_s[...] = jnp.zeros_like(acc_s)

    m = m_s[...]
    gp = jnp.dot(m, wg_ref[...], preferred_element_type=jnp.float32)
    up = jnp.dot(m, wu_ref[...], preferred_element_type=jnp.float32)
    act = gp * (1.0 / (1.0 + jnp.exp(-gp)))
    acc_s[...] += jnp.dot(act * up, wd_ref[...], preferred_element_type=jnp.float32)

    @pl.when(f == NF - 1)
    def _():
        x = x_ref[0]
        out_sel = h1_ref[0] + acc_s[...]
        g = g_ref[0]
        new_ref[0] = x + g * (out_sel - x)


def _mlp_blend(h1, x_sel, gating, ln2, w_gate, w_up, w_down, NF=11):
    B, K, D = x_sel.shape
    FF = w_gate.shape[1]
    fc = FF // NF
    return pl.pallas_call(
        functools.partial(_mlp_body, NF),
        grid=(B, NF),
        in_specs=[
            pl.BlockSpec((1, K, D), lambda b, f: (b, 0, 0)),
            pl.BlockSpec((1, K, D), lambda b, f: (b, 0, 0)),
            pl.BlockSpec((1, K, 1), lambda b, f: (b, 0, 0)),
            pl.BlockSpec((1, D), lambda b, f: (0, 0)),
            pl.BlockSpec((D, fc), lambda b, f: (0, f)),
            pl.BlockSpec((D, fc), lambda b, f: (0, f)),
            pl.BlockSpec((fc, D), lambda b, f: (f, 0)),
        ],
        out_specs=pl.BlockSpec((1, K, D), lambda b, f: (b, 0, 0)),
        out_shape=jax.ShapeDtypeStruct((B, K, D), jnp.float32),
        scratch_shapes=[
            pltpu.VMEM((K, D), jnp.float32),
            pltpu.VMEM((K, D), jnp.float32),
        ],
    )(h1, x_sel, gating, ln2.reshape(1, D), w_gate, w_up, w_down)


def kernel(hidden_states, w_router, wq, wk, wv, wo, w_gate, w_up, w_down, ln1, ln2):
    B, T, D = hidden_states.shape
    H = 16
    Dh = D // H
    K = max(1, int(T * 0.0625))
    AUX_W = 0.01

    hs_flat = hidden_states.reshape(B * T, D)
    out_copy, logits = _copy_logits(hs_flat, w_router)

    idx3, gate3, bce = _topk(logits.reshape(B, 1, T), K)
    idx_flat = idx3.reshape(B * K)
    gating = gate3.reshape(B, K, 1)

    x_sel = _gather(hs_flat, idx_flat).reshape(B, K, D)

    h1 = _attention(x_sel, ln1, wq, wk, wv, wo, H, Dh)
    new_sel = _mlp_blend(h1, x_sel, gating, ln2, w_gate, w_up, w_down)

    new_states = _tc_scatter(out_copy, idx_flat, new_sel.reshape(B * K, D)).reshape(B, T, D)

    aux_loss = bce[0, 0] * (AUX_W / (B * T))
    return new_states, aux_loss


# speedup vs baseline: 1.8760x; 1.8760x over previous
import functools

import jax
import jax.numpy as jnp
from jax import lax
from jax.experimental import pallas as pl
from jax.experimental.pallas import tpu as pltpu
from jax.experimental.pallas import tpu_sc as plsc


def _copy_logits_body(x_ref, wr_ref, out_ref, log_ref):
    x = x_ref[...]
    out_ref[...] = x
    log_ref[...] = jnp.dot(x, wr_ref[...], preferred_element_type=jnp.float32)


def _copy_logits(hs_flat, w_router, rows_per_blk=2048):
    n = hs_flat.shape[0]
    d = hs_flat.shape[1]
    grid = (n // rows_per_blk,)
    return pl.pallas_call(
        _copy_logits_body,
        grid=grid,
        in_specs=[
            pl.BlockSpec((rows_per_blk, d), lambda i: (i, 0)),
            pl.BlockSpec((d, 1), lambda i: (0, 0)),
        ],
        out_specs=[
            pl.BlockSpec((rows_per_blk, d), lambda i: (i, 0)),
            pl.BlockSpec((rows_per_blk, 1), lambda i: (i, 0)),
        ],
        out_shape=[
            jax.ShapeDtypeStruct((n, d), jnp.float32),
            jax.ShapeDtypeStruct((n, 1), jnp.float32),
        ],
    )(hs_flat, w_router)


def _cumsum_lanes(v, upper):
    n = v.shape[1]
    c = upper.shape[0]
    parts = []
    run = jnp.zeros((1, 1), jnp.float32)
    for i in range(n // c):
        blk = v[:, i * c:(i + 1) * c]
        cs = jnp.dot(blk, upper, preferred_element_type=jnp.float32) + run
        run = cs[:, c - 1:c]
        parts.append(cs)
    return jnp.concatenate(parts, axis=1)


def _topk_body(T, K, log_ref, idx_ref, gate_ref, bce_ref):
    b = pl.program_id(0)
    l = log_ref[0]
    bits = lax.bitcast_convert_type(l, jnp.int32)
    key = jnp.where(bits >= 0, bits, bits ^ jnp.int32(0x7FFFFFFF))

    def step(_, lohi):
        lo, hi = lohi
        x = lo ^ hi
        mid = (lo & hi) + (x >> 1) + (x & 1)
        cnt = jnp.sum((key >= mid).astype(jnp.int32))
        big = cnt >= K
        return jnp.where(big, mid, lo), jnp.where(big, hi, mid - 1)

    lo0 = jnp.int32(-2147483647 - 1)
    hi0 = jnp.int32(2147483647)
    t, _ = lax.fori_loop(0, 32, step, (lo0, hi0))

    cnt_gt = jnp.sum((key > t).astype(jnp.float32))
    m_extra = jnp.float32(K) - cnt_gt

    cdim = 1024
    io = lax.broadcasted_iota(jnp.int32, (cdim, cdim), 0)
    ic = lax.broadcasted_iota(jnp.int32, (cdim, cdim), 1)
    upper = (io <= ic).astype(jnp.float32)

    eq = (key == t).astype(jnp.float32)
    cum_eq = _cumsum_lanes(eq, upper)
    mask = jnp.where(key > t, 1.0, 0.0) + eq * (cum_eq <= m_extra)
    pos = _cumsum_lanes(mask, upper) * mask

    iota_f = lax.broadcasted_iota(jnp.int32, (1, T), 1).astype(jnp.float32)
    xt = jnp.concatenate([iota_f, l], axis=0)
    s_row = lax.broadcasted_iota(jnp.int32, (K, cdim), 0).astype(jnp.float32) + 1.0
    sel = jnp.zeros((2, K), jnp.float32)
    for i in range(T // cdim):
        pchunk = pos[:, i * cdim:(i + 1) * cdim]
        onehot = (jnp.broadcast_to(pchunk, (K, cdim)) == s_row).astype(jnp.float32)
        sel = sel + lax.dot_general(
            xt[:, i * cdim:(i + 1) * cdim], onehot,
            (((1,), (1,)), ((), ())), preferred_element_type=jnp.float32,
            precision=lax.Precision.HIGHEST)

    idx_f = sel[0:1, :]
    sel_l = sel[1:2, :]
    idx_ref[...] = (idx_f.astype(jnp.int32) + b * T).reshape(1, 1, K)
    gate_ref[...] = (1.0 / (1.0 + jnp.exp(-sel_l))).reshape(1, 1, K)

    softplus = jnp.maximum(l, 0.0) + jnp.log(1.0 + jnp.exp(-jnp.abs(l)))
    part = (jnp.sum(softplus, keepdims=True)
            - jnp.sum(sel_l, keepdims=True)).reshape(1, 1)

    @pl.when(b == 0)
    def _():
        bce_ref[...] = jnp.zeros((1, 1), jnp.float32)

    bce_ref[...] += part


def _topk(logits3, K):
    B, _, T = logits3.shape
    return pl.pallas_call(
        functools.partial(_topk_body, T, K),
        grid=(B,),
        in_specs=[pl.BlockSpec((1, 1, T), lambda b: (b, 0, 0))],
        out_specs=[
            pl.BlockSpec((1, 1, K), lambda b: (b, 0, 0)),
            pl.BlockSpec((1, 1, K), lambda b: (b, 0, 0)),
            pl.BlockSpec((1, 1), lambda b: (0, 0)),
        ],
        out_shape=[
            jax.ShapeDtypeStruct((B, 1, K), jnp.int32),
            jax.ShapeDtypeStruct((B, 1, K), jnp.float32),
            jax.ShapeDtypeStruct((1, 1), jnp.float32),
        ],
    )(logits3)


def _sc_mesh():
    return plsc.VectorSubcoreMesh(core_axis_name="c", subcore_axis_name="s")


def _sc_wid():
    info = plsc.get_sparse_core_info()
    return lax.axis_index("s") * info.num_cores + lax.axis_index("c")


def _sc_gather(table, idx, d):
    n = idx.shape[0]
    info = plsc.get_sparse_core_info()
    nw = info.num_cores * info.num_subcores
    per = n // nw

    @functools.partial(
        pl.kernel,
        out_type=jax.ShapeDtypeStruct((n, d), jnp.float32),
        mesh=_sc_mesh(),
        scratch_types=[
            pltpu.VMEM((per,), jnp.int32),
            pltpu.VMEM((per, d), jnp.float32),
            pltpu.SemaphoreType.DMA,
        ],
    )
    def k(table_hbm, idx_hbm, out_hbm, idx_v, rows_v, sem):
        base = _sc_wid() * per
        pltpu.sync_copy(idx_hbm.at[pl.ds(base, per)], idx_v)
        pltpu.async_copy(table_hbm.at[idx_v], rows_v, sem).wait()
        pltpu.sync_copy(rows_v, out_hbm.at[pl.ds(base, per)])

    return k(table, idx)


def _gather(table, idx, G=8):
    n, d = table.shape
    m = idx.shape[0]
    table3 = table.reshape(n, 1, d)

    def body(idx_sref, t_ref, out_ref):
        del idx_sref
        out_ref[...] = t_ref[...]

    gspec = pltpu.PrefetchScalarGridSpec(
        num_scalar_prefetch=1,
        grid=(m,),
        in_specs=[pl.BlockSpec((1, 1, d), lambda i, idx_s: (idx_s[i], 0, 0))],
        out_specs=pl.BlockSpec((1, 1, d), lambda i, idx_s: (i, 0, 0)),
    )
    return pl.pallas_call(
        body, grid_spec=gspec,
        out_shape=jax.ShapeDtypeStruct((m, 1, d), jnp.float32),
    )(idx, table3).reshape(m, d)


def _scatter_body(idx_sref, base_ref, rows_ref, out_ref):
    del idx_sref, base_ref
    out_ref[...] = rows_ref[...]


def _tc_scatter(out_copy, idx, rows):
    n, d = out_copy.shape
    m = rows.shape[0]
    gspec = pltpu.PrefetchScalarGridSpec(
        num_scalar_prefetch=1,
        grid=(m,),
        in_specs=[
            pl.BlockSpec(memory_space=pltpu.HBM),
            pl.BlockSpec((1, 1, d), lambda i, idx_s: (i, 0, 0)),
        ],
        out_specs=pl.BlockSpec((1, 1, d), lambda i, idx_s: (idx_s[i], 0, 0)),
    )
    return pl.pallas_call(
        _scatter_body, grid_spec=gspec,
        out_shape=jax.ShapeDtypeStruct((n, 1, d), jnp.float32),
        input_output_aliases={1: 0},
    )(idx, out_copy.reshape(n, 1, d), rows.reshape(m, 1, d))


def _attn_body(H, Dh, x_ref, ln1_ref, wq_ref, wk_ref, wv_ref, wo_ref, h1_ref):
    x = x_ref[0]
    h = x * lax.rsqrt(jnp.mean(x * x, axis=-1, keepdims=True) + 1e-6) * ln1_ref[...]
    K = x.shape[0]
    r_io = lax.broadcasted_iota(jnp.int32, (K, K), 0)
    c_io = lax.broadcasted_iota(jnp.int32, (K, K), 1)
    causal = c_io <= r_io
    scale = 1.0 / (Dh ** 0.5)
    outs = []
    for hd in range(H):
        sl = slice(hd * Dh, (hd + 1) * Dh)
        q = jnp.dot(h, wq_ref[:, sl], preferred_element_type=jnp.float32)
        kk = jnp.dot(h, wk_ref[:, sl], preferred_element_type=jnp.float32)
        v = jnp.dot(h, wv_ref[:, sl], preferred_element_type=jnp.float32)
        s = lax.dot_general(q, kk, (((1,), (1,)), ((), ())),
                            preferred_element_type=jnp.float32) * scale
        s = jnp.where(causal, s, jnp.float32(-1e9))
        m = jnp.max(s, axis=-1, keepdims=True)
        e = jnp.exp(s - m)
        p = e / jnp.sum(e, axis=-1, keepdims=True)
        outs.append(jnp.dot(p, v, preferred_element_type=jnp.float32))
    attn = jnp.concatenate(outs, axis=1)
    h1_ref[0] = x + jnp.dot(attn, wo_ref[...], preferred_element_type=jnp.float32)


def _attention(x_sel, ln1, wq, wk, wv, wo, H, Dh):
    B, K, D = x_sel.shape
    wspec = pl.BlockSpec((D, D), lambda b: (0, 0))
    return pl.pallas_call(
        functools.partial(_attn_body, H, Dh),
        grid=(B,),
        in_specs=[
            pl.BlockSpec((1, K, D), lambda b: (b, 0, 0)),
            pl.BlockSpec((1, D), lambda b: (0, 0)),
            wspec, wspec, wspec, wspec,
        ],
        out_specs=pl.BlockSpec((1, K, D), lambda b: (b, 0, 0)),
        out_shape=jax.ShapeDtypeStruct((B, K, D), jnp.float32),
    )(x_sel, ln1.reshape(1, D), wq, wk, wv, wo)


def _mlp_body(NF, h1_ref, x_ref, g_ref, ln2_ref, wg_ref, wu_ref, wd_ref,
              new_ref, m_s, acc_s):
    f = pl.program_id(1)

    @pl.when(f == 0)
    def _():
        h1 = h1_ref[0]
        m_s[...] = h1 * lax.rsqrt(jnp.mean(h1 * h1, axis=-1, keepdims=True)
                                  + 1e-6) * ln2_ref[...]
        acc_s[...] = jnp.zeros_like(acc_s)

    m = m_s[...]
    gp = jnp.dot(m, wg_ref[...], preferred_element_type=jnp.float32)
    up = jnp.dot(m, wu_ref[...], preferred_element_type=jnp.float32)
    act = gp * (1.0 / (1.0 + jnp.exp(-gp)))
    acc_s[...] += jnp.dot(act * up, wd_ref[...], preferred_element_type=jnp.float32)

    @pl.when(f == NF - 1)
    def _():
        x = x_ref[0]
        out_sel = h1_ref[0] + acc_s[...]
        g = g_ref[0]
        new_ref[0] = x + g * (out_sel - x)


def _mlp_blend(h1, x_sel, gating, ln2, w_gate, w_up, w_down, NF=11):
    B, K, D = x_sel.shape
    FF = w_gate.shape[1]
    fc = FF // NF
    return pl.pallas_call(
        functools.partial(_mlp_body, NF),
        grid=(B, NF),
        in_specs=[
            pl.BlockSpec((1, K, D), lambda b, f: (b, 0, 0)),
            pl.BlockSpec((1, K, D), lambda b, f: (b, 0, 0)),
            pl.BlockSpec((1, K, 1), lambda b, f: (b, 0, 0)),
            pl.BlockSpec((1, D), lambda b, f: (0, 0)),
            pl.BlockSpec((D, fc), lambda b, f: (0, f)),
            pl.BlockSpec((D, fc), lambda b, f: (0, f)),
            pl.BlockSpec((fc, D), lambda b, f: (f, 0)),
        ],
        out_specs=pl.BlockSpec((1, K, D), lambda b, f: (b, 0, 0)),
        out_shape=jax.ShapeDtypeStruct((B, K, D), jnp.float32),
        scratch_shapes=[
            pltpu.VMEM((K, D), jnp.float32),
            pltpu.VMEM((K, D), jnp.float32),
        ],
    )(h1, x_sel, gating, ln2.reshape(1, D), w_gate, w_up, w_down)


def kernel(hidden_states, w_router, wq, wk, wv, wo, w_gate, w_up, w_down, ln1, ln2):
    B, T, D = hidden_states.shape
    H = 16
    Dh = D // H
    K = max(1, int(T * 0.0625))
    AUX_W = 0.01

    hs_flat = hidden_states.reshape(B * T, D)
    out_copy, logits = _copy_logits(hs_flat, w_router)

    idx3, gate3, bce = _topk(logits.reshape(B, 1, T), K)
    idx_flat = idx3.reshape(B * K)
    gating = gate3.reshape(B, K, 1)

    x_sel = _sc_gather(hs_flat, idx_flat, D).reshape(B, K, D)

    h1 = _attention(x_sel, ln1, wq, wk, wv, wo, H, Dh)
    new_sel = _mlp_blend(h1, x_sel, gating, ln2, w_gate, w_up, w_down)

    new_states = _tc_scatter(out_copy, idx_flat, new_sel.reshape(B * K, D)).reshape(B, T, D)

    aux_loss = bce[0, 0] * (AUX_W / (B * T))
    return new_states, aux_loss
